# Initial kernel scaffold; baseline (speedup 1.0000x reference)
import functools

import jax
import jax.numpy as jnp
from jax import lax
from jax.experimental import pallas as pl
from jax.experimental.pallas import tpu as pltpu
from jax.experimental.pallas import tpu_sc as plsc

B, L, D = 16384, 200, 32
BL = B * L
NC, NS = 2, 16
NW = NC * NS
PER_W = BL // NW
SEQS_PER_CHUNK = 4
C = SEQS_PER_CHUNK * L
NCHUNK = PER_W // C
GSUB = 128


@functools.partial(
    pl.kernel,
    mesh=plsc.VectorSubcoreMesh(core_axis_name="c", subcore_axis_name="s"),
    out_type=jax.ShapeDtypeStruct((BL, D), jnp.float32),
    scratch_types=[
        pltpu.VMEM((C,), jnp.int32),
        pltpu.VMEM((C, D), jnp.float32),
        pltpu.VMEM((L, D), jnp.float32),
        pltpu.SemaphoreType.DMA,
    ],
)
def _embed(x_hbm, tab_hbm, pos_hbm, out_hbm, idx_v, rows_v, pos_v, gsem):
    wid = lax.axis_index("s") * NC + lax.axis_index("c")
    base = wid * PER_W
    pltpu.sync_copy(pos_hbm, pos_v)

    def chunk(g, carry):
        cb = pl.multiple_of(base + g * C, 8)
        pltpu.sync_copy(x_hbm.at[pl.ds(cb, C)], idx_v)
        cps = []
        for off in range(0, C, GSUB):
            n = min(GSUB, C - off)
            cps.append(pltpu.async_copy(
                tab_hbm.at[idx_v.at[pl.ds(off, n)]],
                rows_v.at[pl.ds(off, n)], gsem))
        for cp in cps:
            cp.wait()

        def add_seq(r, carry2):
            for k in range(SEQS_PER_CHUNK):
                for col in (0, 16):
                    rows_v[k * L + r, pl.ds(col, 16)] += pos_v[r, pl.ds(col, 16)]
            return carry2

        lax.fori_loop(0, L, add_seq, 0)
        pltpu.sync_copy(rows_v, out_hbm.at[pl.ds(cb, C)])
        return carry

    lax.fori_loop(0, NCHUNK, chunk, 0)


def kernel(x, window_table, pos_table):
    x_flat = x.reshape(-1).astype(jnp.int32)
    out = _embed(x_flat, window_table, pos_table)
    return out.reshape(B, L, D)


# speedup vs baseline: 4.4811x; 4.4811x over previous
import functools

import jax
import jax.numpy as jnp
from jax import lax
from jax.experimental import pallas as pl
from jax.experimental.pallas import tpu as pltpu
from jax.experimental.pallas import tpu_sc as plsc

B, L, D = 16384, 200, 32
BL = B * L
NC, NS = 2, 16
NW = NC * NS
PER_W = BL // NW
SEQS_PER_CHUNK = 4
C = SEQS_PER_CHUNK * L
NCHUNK = PER_W // C
GSUB = 128


@functools.partial(
    pl.kernel,
    mesh=plsc.VectorSubcoreMesh(core_axis_name="c", subcore_axis_name="s"),
    out_type=jax.ShapeDtypeStruct((BL, D), jnp.float32),
    compiler_params=pltpu.CompilerParams(use_tc_tiling_on_sc=False),
    scratch_types=[
        pltpu.VMEM((C,), jnp.int32),
        pltpu.VMEM((C, D), jnp.float32),
        pltpu.VMEM((L, D), jnp.float32),
        pltpu.SemaphoreType.DMA,
    ],
)
def _embed(x_hbm, tab_hbm, pos_hbm, out_hbm, idx_v, rows_v, pos_v, gsem):
    wid = lax.axis_index("s") * NC + lax.axis_index("c")
    base = wid * PER_W
    pltpu.sync_copy(pos_hbm, pos_v)

    def chunk(g, carry):
        cb = pl.multiple_of(base + g * C, 8)
        pltpu.sync_copy(x_hbm.at[pl.ds(cb, C)], idx_v)
        cps = []
        for off in range(0, C, GSUB):
            n = min(GSUB, C - off)
            cps.append(pltpu.async_copy(
                tab_hbm.at[idx_v.at[pl.ds(off, n)]],
                rows_v.at[pl.ds(off, n)], gsem))
        for cp in cps:
            cp.wait()

        def add_seq(r, carry2):
            for k in range(SEQS_PER_CHUNK):
                for col in (0, 16):
                    rows_v[k * L + r, pl.ds(col, 16)] += pos_v[r, pl.ds(col, 16)]
            return carry2

        lax.fori_loop(0, L, add_seq, 0)
        pltpu.sync_copy(rows_v, out_hbm.at[pl.ds(cb, C)])
        return carry

    lax.fori_loop(0, NCHUNK, chunk, 0)


def kernel(x, window_table, pos_table):
    x_flat = x.reshape(-1).astype(jnp.int32)
    out = _embed(x_flat, window_table, pos_table)
    return out.reshape(B, L, D)


# speedup vs baseline: 4.5716x; 1.0202x over previous
import functools

import jax
import jax.numpy as jnp
from jax import lax
from jax.experimental import pallas as pl
from jax.experimental.pallas import tpu as pltpu
from jax.experimental.pallas import tpu_sc as plsc

B, L, D = 16384, 200, 32
BL = B * L
NC, NS = 2, 16
NW = NC * NS
PER_W = BL // NW
SEQS_PER_CHUNK = 2
C = SEQS_PER_CHUNK * L
NCHUNK = PER_W // C
GSUB = 128
NBUF = 4


@functools.partial(
    pl.kernel,
    mesh=plsc.VectorSubcoreMesh(core_axis_name="c", subcore_axis_name="s"),
    out_type=jax.ShapeDtypeStruct((BL, D), jnp.float32),
    compiler_params=pltpu.CompilerParams(use_tc_tiling_on_sc=False),
    scratch_types=(
        [pltpu.VMEM((C,), jnp.int32) for _ in range(NBUF)]
        + [pltpu.VMEM((C, D), jnp.float32) for _ in range(NBUF)]
        + [pltpu.VMEM((L, D), jnp.float32)]
        + [pltpu.SemaphoreType.DMA for _ in range(2 * NBUF)]
    ),
)
def _embed(x_hbm, tab_hbm, pos_hbm, out_hbm, *scratch):
    idx_b = scratch[0:NBUF]
    rows_b = scratch[NBUF:2 * NBUF]
    pos_v = scratch[2 * NBUF]
    gsem_b = scratch[2 * NBUF + 1:3 * NBUF + 1]
    ssem_b = scratch[3 * NBUF + 1:4 * NBUF + 1]

    wid = lax.axis_index("s") * NC + lax.axis_index("c")
    base = wid * PER_W

    pltpu.sync_copy(pos_hbm, pos_v)

    def gather_pieces(b):
        for off in range(0, C, GSUB):
            n = min(GSUB, C - off)
            yield (tab_hbm.at[idx_b[b].at[pl.ds(off, n)]],
                   rows_b[b].at[pl.ds(off, n)], gsem_b[b])

    def fire_gather(g, b):
        cb = pl.multiple_of(base + g * C, 8)
        pltpu.sync_copy(x_hbm.at[pl.ds(cb, C)], idx_b[b])
        for src, dst, sem in gather_pieces(b):
            pltpu.async_copy(src, dst, sem)

    def wait_gather(b):
        for src, dst, sem in gather_pieces(b):
            pltpu.make_async_copy(src, dst, sem).wait()

    def fire_store(g, b):
        cb = pl.multiple_of(base + g * C, 8)
        pltpu.async_copy(rows_b[b], out_hbm.at[pl.ds(cb, C)], ssem_b[b])

    def wait_store(b):
        pltpu.make_async_copy(rows_b[b], out_hbm.at[pl.ds(0, C)],
                              ssem_b[b]).wait()

    def add_pos(b):
        rows = rows_b[b]

        def add_seq(r, carry):
            for k in range(SEQS_PER_CHUNK):
                for col in (0, 16):
                    rows[k * L + r, pl.ds(col, 16)] += pos_v[r, pl.ds(col, 16)]
            return carry

        lax.fori_loop(0, L, add_seq, 0)

    for g in range(NBUF - 1):
        fire_gather(g, g)
    wait_gather(0)
    fire_gather(NBUF - 1, NBUF - 1)
    add_pos(0)
    fire_store(0, 0)

    n_steady = NCHUNK - NBUF
    assert n_steady % NBUF == 0

    def steady(s, carry):
        for u in range(NBUF):
            g = 1 + s * NBUF + u
            b = (1 + u) % NBUF
            wait_gather(b)
            wait_store((b + NBUF - 1) % NBUF)
            fire_gather(g + NBUF - 1, (b + NBUF - 1) % NBUF)
            add_pos(b)
            fire_store(g, b)
        return carry

    lax.fori_loop(0, n_steady // NBUF, steady, 0)

    for k in range(NBUF - 1):
        g = NCHUNK - (NBUF - 1) + k
        b = g % NBUF
        wait_gather(b)
        add_pos(b)
        fire_store(g, b)
    for b in range(NBUF):
        wait_store(b)


def kernel(x, window_table, pos_table):
    x_flat = x.reshape(-1).astype(jnp.int32)
    out = _embed(x_flat, window_table, pos_table)
    return out.reshape(B, L, D)
